# baseline (speedup 1.0000x reference)
import functools

import jax
import jax.numpy as jnp
import numpy as np
from jax.experimental import pallas as pl
from jax.experimental.pallas import tpu as pltpu

SA_SPECS = [
    (1024, (0.05, 0.1), (16, 32), ((3, 16, 16, 32), (3, 32, 32, 64))),
    (256, (0.1, 0.2), (16, 32), ((99, 64, 64, 128), (99, 64, 96, 128))),
    (64, (0.2, 0.4), (16, 32), ((259, 128, 196, 256), (259, 128, 196, 256))),
    (16, (0.4, 0.8), (16, 32), ((515, 256, 256, 512), (515, 256, 384, 512))),
]
B, N = 4, 8192


def _index_points(points, idx):
    return jax.vmap(lambda p, i: p[i])(points, idx)


def _fps(xyz, npoint):
    xyz = jax.lax.stop_gradient(xyz)
    b, n, _ = xyz.shape
    def step(carry, _):
        dists, farthest = carry
        centroid = _index_points(xyz, farthest[:, None])
        d = jnp.sum((xyz - centroid) ** 2, axis=-1)
        dists = jnp.minimum(dists, d)
        new_far = jnp.argmax(dists, axis=-1).astype(jnp.int32)
        return (dists, new_far), farthest
    init = (jnp.full((b, n), 1e10, jnp.float32), jnp.zeros((b,), jnp.int32))
    _, idxs = jax.lax.scan(step, init, None, length=npoint)
    return jnp.transpose(idxs)


def _ball_query(radius, nsample, xyz, new_xyz):
    b, n, _ = xyz.shape
    s = new_xyz.shape[1]
    d2 = jnp.sum((new_xyz[:, :, None, :] - xyz[:, None, :, :]) ** 2, axis=-1)
    gidx = jnp.broadcast_to(jnp.arange(n, dtype=jnp.int32), (b, s, n))
    gidx = jnp.where(d2 > radius * radius, n, gidx)
    gidx = jnp.sort(gidx, axis=-1)[:, :, :nsample]
    first = gidx[:, :, :1]
    gidx = jnp.where(gidx == n, jnp.broadcast_to(first, gidx.shape), gidx)
    return gidx


def _fps_kernel(npoint, n, b, coordsT_ref, out_ref, dists_ref):
    X = coordsT_ref[:, 0, :]
    Y = coordsT_ref[:, 1, :]
    Z = coordsT_ref[:, 2, :]
    iota = jax.lax.broadcasted_iota(jnp.int32, (b, n), 1)
    dists_ref[...] = jnp.full((b, n), 1e10, jnp.float32)

    def step(j, far):
        out_ref[pl.ds(j, 1), :] = jnp.transpose(far)
        eq = iota == far
        cx = jnp.sum(jnp.where(eq, X, 0.0), axis=1, keepdims=True)
        cy = jnp.sum(jnp.where(eq, Y, 0.0), axis=1, keepdims=True)
        cz = jnp.sum(jnp.where(eq, Z, 0.0), axis=1, keepdims=True)
        dx = X - cx
        dy = Y - cy
        dz = Z - cz
        d = dx * dx + dy * dy + dz * dz
        dists = jnp.minimum(dists_ref[...], d)
        dists_ref[...] = dists
        m = jnp.max(dists, axis=1, keepdims=True)
        return jnp.min(jnp.where(dists == m, iota, n), axis=1, keepdims=True)

    jax.lax.fori_loop(0, npoint, step, jnp.zeros((b, 1), jnp.int32))


def _fps_pallas(xyz, npoint):
    b, n, _ = xyz.shape
    coordsT = jnp.transpose(xyz, (0, 2, 1))
    fn = functools.partial(_fps_kernel, npoint, n, b)
    out = pl.pallas_call(
        fn,
        grid=(1,),
        in_specs=[pl.BlockSpec((b, 3, n), lambda i: (0, 0, 0))],
        out_specs=pl.BlockSpec((npoint, b), lambda i: (0, 0)),
        out_shape=jax.ShapeDtypeStruct((npoint, b), jnp.int32),
        scratch_shapes=[pltpu.VMEM((b, n), jnp.float32)],
    )(coordsT)
    return jnp.transpose(out)


def _bq_kernel(n_actual, radius2, nsample, new_ref, xyzT_ref, out_ref):
    new = new_ref[0]
    cx, cy, cz = new[:, 0:1], new[:, 1:2], new[:, 2:3]
    X = xyzT_ref[0]
    dx = cx - X[0:1, :]
    dy = cy - X[1:2, :]
    dz = cz - X[2:3, :]
    d2 = dx * dx + dy * dy + dz * dz
    iota = jax.lax.broadcasted_iota(jnp.int32, d2.shape, 1)
    key = jnp.where(d2 > radius2, n_actual, iota)
    cols = []
    for _ in range(nsample):
        m = jnp.min(key, axis=1, keepdims=True)
        cols.append(m)
        key = jnp.where(key == m, n_actual, key)
    out = jnp.concatenate(cols, axis=1)
    first = out[:, 0:1]
    out_ref[0] = jnp.where(out == n_actual, first, out)


def _ball_query_pallas(radius, nsample, xyz, new_xyz):
    b, n, _ = xyz.shape
    s = new_xyz.shape[1]
    xyzT = jnp.transpose(xyz, (0, 2, 1))
    bs = min(128, s)
    r2 = np.float32(radius * radius)
    fn = functools.partial(_bq_kernel, n, r2, nsample)
    return pl.pallas_call(
        fn,
        grid=(b, s // bs),
        in_specs=[pl.BlockSpec((1, bs, 3), lambda i, j: (i, j, 0)),
                  pl.BlockSpec((1, 3, n), lambda i, j: (i, 0, 0))],
        out_specs=pl.BlockSpec((1, bs, nsample), lambda i, j: (i, j, 0)),
        out_shape=jax.ShapeDtypeStruct((b, s, nsample), jnp.int32),
    )(new_xyz, xyzT)


def _interp_kernel(s_actual, unk_ref, knownT_ref, feats_ref, out_ref):
    unk = unk_ref[0]
    Kt = knownT_ref[0]
    dx = unk[:, 0:1] - Kt[0:1, :]
    dy = unk[:, 1:2] - Kt[1:2, :]
    dz = unk[:, 2:3] - Kt[2:3, :]
    d2 = dx * dx + dy * dy + dz * dz
    iota = jax.lax.broadcasted_iota(jnp.int32, d2.shape, 1)
    dvals, idxs = [], []
    work = d2
    for _ in range(3):
        m = jnp.min(work, axis=1, keepdims=True)
        ix = jnp.min(jnp.where(work == m, iota, s_actual), axis=1, keepdims=True)
        dvals.append(m)
        idxs.append(ix)
        work = jnp.where(iota == ix, jnp.float32(jnp.inf), work)
    ws = [1.0 / (m + 1e-8) for m in dvals]
    wsum = ws[0] + ws[1] + ws[2]
    W = jnp.zeros_like(d2)
    for wk, ix in zip(ws, idxs):
        W = W + jnp.where(iota == ix, wk / wsum, 0.0)
    out_ref[0] = jax.lax.dot_general(
        W, feats_ref[0], (((1,), (0,)), ((), ())),
        preferred_element_type=jnp.float32,
        precision=jax.lax.Precision.HIGHEST)


def _fp_interp_pallas(unknown_xyz, known_xyz, known_feats):
    b, nu, _ = unknown_xyz.shape
    s = known_xyz.shape[1]
    c = known_feats.shape[2]
    knownT = jnp.transpose(known_xyz, (0, 2, 1))
    bu = min(256, nu)
    fn = functools.partial(_interp_kernel, s)
    return pl.pallas_call(
        fn,
        grid=(b, nu // bu),
        in_specs=[pl.BlockSpec((1, bu, 3), lambda i, j: (i, j, 0)),
                  pl.BlockSpec((1, 3, s), lambda i, j: (i, 0, 0)),
                  pl.BlockSpec((1, s, c), lambda i, j: (i, 0, 0))],
        out_specs=pl.BlockSpec((1, bu, c), lambda i, j: (i, j, 0)),
        out_shape=jax.ShapeDtypeStruct((b, nu, c), jnp.float32),
    )(unknown_xyz, knownT, known_feats)


def _mlp_kernel(nlayers, pool_k, x_ref, *refs):
    out_ref = refs[-1]
    x = x_ref[0]
    for li in range(nlayers):
        w = refs[2 * li][...]
        bias = refs[2 * li + 1][...]
        x = jax.lax.dot_general(x, w, (((1,), (0,)), ((), ())),
                                preferred_element_type=jnp.float32,
                                precision=jax.lax.Precision.HIGHEST)
        x = jnp.maximum(x + bias, 0.0)
    if pool_k is not None:
        m, c = x.shape
        x = jnp.max(x.reshape(m // pool_k, pool_k, c), axis=1)
    out_ref[0] = x


def _mlp_pallas(x, layers, pool_k=None, m_blk=512):
    b, m, cin = x.shape
    mb = min(m_blk, m)
    cout = layers[-1][0].shape[0]
    mout = m // pool_k if pool_k is not None else m
    mob = mb // pool_k if pool_k is not None else mb
    wb = []
    in_specs = [pl.BlockSpec((1, mb, cin), lambda i, j: (i, j, 0))]
    for W, bias in layers:
        wT = jnp.transpose(W)
        wb.extend([wT, bias[None, :]])
        in_specs.append(pl.BlockSpec(wT.shape, lambda i, j: (0, 0)))
        in_specs.append(pl.BlockSpec((1, bias.shape[0]), lambda i, j: (0, 0)))
    fn = functools.partial(_mlp_kernel, len(layers), pool_k)
    return pl.pallas_call(
        fn,
        grid=(b, m // mb),
        in_specs=in_specs,
        out_specs=pl.BlockSpec((1, mob, cout), lambda i, j: (i, j, 0)),
        out_shape=jax.ShapeDtypeStruct((b, mout, cout), jnp.float32),
    )(x, *wb)


def _sa_msg(xyz, feats, npoint, radii, nsamples, scale_params):
    fidx = _fps_pallas(jax.lax.stop_gradient(xyz), npoint)
    new_xyz = _index_points(xyz, fidx)
    sg_xyz = jax.lax.stop_gradient(xyz)
    sg_new = jax.lax.stop_gradient(new_xyz)
    outs = []
    s = sg_new.shape[1]
    for radius, nsample, layers in zip(radii, nsamples, scale_params):
        idx = _ball_query_pallas(radius, nsample, sg_xyz, sg_new)
        g = _index_points(xyz, idx) - new_xyz[:, :, None, :]
        if feats is not None:
            g = jnp.concatenate([g, _index_points(feats, idx)], axis=-1)
        b, _, _, cin = g.shape
        mb = min(128, s) * nsample
        out = _mlp_pallas(g.reshape(b, s * nsample, cin), layers,
                          pool_k=nsample, m_blk=mb)
        outs.append(out)
    return new_xyz, jnp.concatenate(outs, axis=-1)


def _fp_module(unknown_xyz, known_xyz, unk_feats, known_feats, layers):
    interp = _fp_interp_pallas(jax.lax.stop_gradient(unknown_xyz),
                               jax.lax.stop_gradient(known_xyz), known_feats)
    x = interp if unk_feats is None else jnp.concatenate([interp, unk_feats], axis=-1)
    return _mlp_pallas(x, layers, m_blk=512)


def _final_kernel(feat_ref, tout_ref, fmax_ref):
    f = feat_ref[0]
    tout_ref[0] = jnp.transpose(f, (1, 0))
    fmax_ref[0] = jnp.max(f, axis=0, keepdims=True)


def _final_stage(feat):
    b, n, c = feat.shape
    tout, fmax = pl.pallas_call(
        _final_kernel,
        grid=(b,),
        in_specs=[pl.BlockSpec((1, n, c), lambda i: (i, 0, 0))],
        out_specs=[pl.BlockSpec((1, c, n), lambda i: (i, 0, 0)),
                   pl.BlockSpec((1, 1, c), lambda i: (i, 0, 0))],
        out_shape=[jax.ShapeDtypeStruct((b, c, n), feat.dtype),
                   jax.ShapeDtypeStruct((b, 1, c), feat.dtype)],
    )(feat)
    return tout, fmax[:, 0, :]


def kernel(pointcloud, params):
    xyz = pointcloud[..., :3]
    l_xyz, l_feat = [xyz], [None]
    for (npoint, radii, nsamples, _), sp in zip(SA_SPECS, params['sa']):
        nx, nf = _sa_msg(l_xyz[-1], l_feat[-1], npoint, radii, nsamples, sp)
        l_xyz.append(nx)
        l_feat.append(nf)
    sa_glob = jnp.max(l_feat[-1], axis=1)
    for i in range(-1, -5, -1):
        l_feat[i - 1] = _fp_module(l_xyz[i - 1], l_xyz[i], l_feat[i - 1], l_feat[i], params['fp'][i])
    feat = l_feat[0]
    tfeat, fmax = _final_stage(feat)
    global_feat = jnp.concatenate([fmax, sa_glob], axis=-1)
    return tfeat, global_feat


# speedup vs baseline: 2.4528x; 2.4528x over previous
import functools

import jax
import jax.numpy as jnp
import numpy as np
from jax.experimental import pallas as pl
from jax.experimental.pallas import tpu as pltpu
from jax.experimental.pallas import tpu_sc as plsc

SA_SPECS = [
    (1024, (0.05, 0.1), (16, 32), ((3, 16, 16, 32), (3, 32, 32, 64))),
    (256, (0.1, 0.2), (16, 32), ((99, 64, 64, 128), (99, 64, 96, 128))),
    (64, (0.2, 0.4), (16, 32), ((259, 128, 196, 256), (259, 128, 196, 256))),
    (16, (0.4, 0.8), (16, 32), ((515, 256, 256, 512), (515, 256, 384, 512))),
]
B, N = 4, 8192


def _index_points(points, idx):
    return jax.vmap(lambda p, i: p[i])(points, idx)


def _fps(xyz, npoint):
    xyz = jax.lax.stop_gradient(xyz)
    b, n, _ = xyz.shape
    def step(carry, _):
        dists, farthest = carry
        centroid = _index_points(xyz, farthest[:, None])
        d = jnp.sum((xyz - centroid) ** 2, axis=-1)
        dists = jnp.minimum(dists, d)
        new_far = jnp.argmax(dists, axis=-1).astype(jnp.int32)
        return (dists, new_far), farthest
    init = (jnp.full((b, n), 1e10, jnp.float32), jnp.zeros((b,), jnp.int32))
    _, idxs = jax.lax.scan(step, init, None, length=npoint)
    return jnp.transpose(idxs)


def _ball_query(radius, nsample, xyz, new_xyz):
    b, n, _ = xyz.shape
    s = new_xyz.shape[1]
    d2 = jnp.sum((new_xyz[:, :, None, :] - xyz[:, None, :, :]) ** 2, axis=-1)
    gidx = jnp.broadcast_to(jnp.arange(n, dtype=jnp.int32), (b, s, n))
    gidx = jnp.where(d2 > radius * radius, n, gidx)
    gidx = jnp.sort(gidx, axis=-1)[:, :, :nsample]
    first = gidx[:, :, :1]
    gidx = jnp.where(gidx == n, jnp.broadcast_to(first, gidx.shape), gidx)
    return gidx


def _fps_kernel(npoint, n, b, coordsT_ref, out_ref, dists_ref):
    X = coordsT_ref[:, 0, :]
    Y = coordsT_ref[:, 1, :]
    Z = coordsT_ref[:, 2, :]
    iota = jax.lax.broadcasted_iota(jnp.int32, (b, n), 1)
    dists_ref[...] = jnp.full((b, n), 1e10, jnp.float32)

    def step(j, far):
        out_ref[pl.ds(j, 1), :] = jnp.transpose(far)
        eq = iota == far
        cx = jnp.sum(jnp.where(eq, X, 0.0), axis=1, keepdims=True)
        cy = jnp.sum(jnp.where(eq, Y, 0.0), axis=1, keepdims=True)
        cz = jnp.sum(jnp.where(eq, Z, 0.0), axis=1, keepdims=True)
        dx = X - cx
        dy = Y - cy
        dz = Z - cz
        d = dx * dx + dy * dy + dz * dz
        dists = jnp.minimum(dists_ref[...], d)
        dists_ref[...] = dists
        m = jnp.max(dists, axis=1, keepdims=True)
        return jnp.min(jnp.where(dists == m, iota, n), axis=1, keepdims=True)

    jax.lax.fori_loop(0, npoint, step, jnp.zeros((b, 1), jnp.int32))


def _fps_pallas(xyz, npoint):
    b, n, _ = xyz.shape
    coordsT = jnp.transpose(xyz, (0, 2, 1))
    fn = functools.partial(_fps_kernel, npoint, n, b)
    out = pl.pallas_call(
        fn,
        grid=(1,),
        in_specs=[pl.BlockSpec((b, 3, n), lambda i: (0, 0, 0))],
        out_specs=pl.BlockSpec((npoint, b), lambda i: (0, 0)),
        out_shape=jax.ShapeDtypeStruct((npoint, b), jnp.int32),
        scratch_shapes=[pltpu.VMEM((b, n), jnp.float32)],
    )(coordsT)
    return jnp.transpose(out)


def _bq_kernel(n_actual, radius2, nsample, new_ref, xyzT_ref, out_ref):
    new = new_ref[0]
    cx, cy, cz = new[:, 0:1], new[:, 1:2], new[:, 2:3]
    X = xyzT_ref[0]
    dx = cx - X[0:1, :]
    dy = cy - X[1:2, :]
    dz = cz - X[2:3, :]
    d2 = dx * dx + dy * dy + dz * dz
    iota = jax.lax.broadcasted_iota(jnp.int32, d2.shape, 1)
    key = jnp.where(d2 > radius2, n_actual, iota)
    cols = []
    for _ in range(nsample):
        m = jnp.min(key, axis=1, keepdims=True)
        cols.append(m)
        key = jnp.where(key == m, n_actual, key)
    out = jnp.concatenate(cols, axis=1)
    first = out[:, 0:1]
    out_ref[0] = jnp.where(out == n_actual, first, out)


def _ball_query_pallas(radius, nsample, xyz, new_xyz):
    b, n, _ = xyz.shape
    s = new_xyz.shape[1]
    xyzT = jnp.transpose(xyz, (0, 2, 1))
    bs = min(128, s)
    r2 = np.float32(radius * radius)
    fn = functools.partial(_bq_kernel, n, r2, nsample)
    return pl.pallas_call(
        fn,
        grid=(b, s // bs),
        in_specs=[pl.BlockSpec((1, bs, 3), lambda i, j: (i, j, 0)),
                  pl.BlockSpec((1, 3, n), lambda i, j: (i, 0, 0))],
        out_specs=pl.BlockSpec((1, bs, nsample), lambda i, j: (i, j, 0)),
        out_shape=jax.ShapeDtypeStruct((b, s, nsample), jnp.int32),
    )(new_xyz, xyzT)


def _interp_kernel(s_actual, unk_ref, knownT_ref, feats_ref, out_ref):
    unk = unk_ref[0]
    Kt = knownT_ref[0]
    dx = unk[:, 0:1] - Kt[0:1, :]
    dy = unk[:, 1:2] - Kt[1:2, :]
    dz = unk[:, 2:3] - Kt[2:3, :]
    d2 = dx * dx + dy * dy + dz * dz
    iota = jax.lax.broadcasted_iota(jnp.int32, d2.shape, 1)
    dvals, idxs = [], []
    work = d2
    for _ in range(3):
        m = jnp.min(work, axis=1, keepdims=True)
        ix = jnp.min(jnp.where(work == m, iota, s_actual), axis=1, keepdims=True)
        dvals.append(m)
        idxs.append(ix)
        work = jnp.where(iota == ix, jnp.float32(jnp.inf), work)
    ws = [1.0 / (m + 1e-8) for m in dvals]
    wsum = ws[0] + ws[1] + ws[2]
    W = jnp.zeros_like(d2)
    for wk, ix in zip(ws, idxs):
        W = W + jnp.where(iota == ix, wk / wsum, 0.0)
    out_ref[0] = jax.lax.dot_general(
        W, feats_ref[0], (((1,), (0,)), ((), ())),
        preferred_element_type=jnp.float32,
        precision=jax.lax.Precision.HIGHEST)


def _fp_interp_pallas(unknown_xyz, known_xyz, known_feats):
    b, nu, _ = unknown_xyz.shape
    s = known_xyz.shape[1]
    c = known_feats.shape[2]
    knownT = jnp.transpose(known_xyz, (0, 2, 1))
    bu = min(256, nu)
    fn = functools.partial(_interp_kernel, s)
    return pl.pallas_call(
        fn,
        grid=(b, nu // bu),
        in_specs=[pl.BlockSpec((1, bu, 3), lambda i, j: (i, j, 0)),
                  pl.BlockSpec((1, 3, s), lambda i, j: (i, 0, 0)),
                  pl.BlockSpec((1, s, c), lambda i, j: (i, 0, 0))],
        out_specs=pl.BlockSpec((1, bu, c), lambda i, j: (i, j, 0)),
        out_shape=jax.ShapeDtypeStruct((b, nu, c), jnp.float32),
    )(unknown_xyz, knownT, known_feats)


def _sc_gather_rows(table, idx_flat):
    m = idx_flat.shape[0]
    d = table.shape[1]
    info = plsc.get_sparse_core_info()
    ncores = info.num_cores
    nw = ncores * info.num_subcores
    bpw = m // nw
    chunk = min(128, bpw)
    nchunk = bpw // chunk
    idx3 = idx_flat.reshape(nw, nchunk, chunk)

    @functools.partial(
        pl.kernel,
        mesh=plsc.VectorSubcoreMesh(core_axis_name="c", subcore_axis_name="s"),
        out_type=jax.ShapeDtypeStruct((m, d), jnp.float32),
        compiler_params=pltpu.CompilerParams(use_tc_tiling_on_sc=False),
        scratch_types=[pltpu.VMEM((nchunk, chunk), jnp.int32),
                       pltpu.VMEM((bpw, d), jnp.float32),
                       pltpu.SemaphoreType.DMA],
    )
    def gk(table_hbm, idx_hbm, out_hbm, idx_v, rows_v, sem):
        wid = jax.lax.axis_index("s") * ncores + jax.lax.axis_index("c")
        pltpu.sync_copy(idx_hbm.at[wid], idx_v)

        def start(j, carry):
            pltpu.make_async_copy(table_hbm.at[idx_v.at[j]],
                                  rows_v.at[pl.ds(j * chunk, chunk)], sem).start()
            return carry

        jax.lax.fori_loop(0, nchunk, start, 0)

        def drain(j, carry):
            pltpu.make_async_copy(table_hbm.at[idx_v.at[j]],
                                  rows_v.at[pl.ds(j * chunk, chunk)], sem).wait()
            return carry

        jax.lax.fori_loop(0, nchunk, drain, 0)
        pltpu.sync_copy(rows_v, out_hbm.at[pl.ds(wid * bpw, bpw)])

    return gk(table, idx3)


def _pad16(x):
    c = x.shape[-1]
    pad = (-c) % 16
    if pad:
        x = jnp.concatenate([x, jnp.zeros(x.shape[:-1] + (pad,), x.dtype)], axis=-1)
    return x


def _mlp_kernel(nlayers, pool_k, x_ref, *refs):
    out_ref = refs[-1]
    x = x_ref[0]
    for li in range(nlayers):
        w = refs[2 * li][...]
        bias = refs[2 * li + 1][...]
        x = jax.lax.dot_general(x, w, (((1,), (0,)), ((), ())),
                                preferred_element_type=jnp.float32,
                                precision=jax.lax.Precision.HIGHEST)
        x = jnp.maximum(x + bias, 0.0)
    if pool_k is not None:
        m, c = x.shape
        x = jnp.max(x.reshape(m // pool_k, pool_k, c), axis=1)
    out_ref[0] = x


def _mlp_pallas(x, layers, pool_k=None, m_blk=512):
    b, m, cin = x.shape
    mb = min(m_blk, m)
    cout = layers[-1][0].shape[0]
    mout = m // pool_k if pool_k is not None else m
    mob = mb // pool_k if pool_k is not None else mb
    wb = []
    in_specs = [pl.BlockSpec((1, mb, cin), lambda i, j: (i, j, 0))]
    for W, bias in layers:
        wT = jnp.transpose(W)
        wb.extend([wT, bias[None, :]])
        in_specs.append(pl.BlockSpec(wT.shape, lambda i, j: (0, 0)))
        in_specs.append(pl.BlockSpec((1, bias.shape[0]), lambda i, j: (0, 0)))
    fn = functools.partial(_mlp_kernel, len(layers), pool_k)
    return pl.pallas_call(
        fn,
        grid=(b, m // mb),
        in_specs=in_specs,
        out_specs=pl.BlockSpec((1, mob, cout), lambda i, j: (i, j, 0)),
        out_shape=jax.ShapeDtypeStruct((b, mout, cout), jnp.float32),
    )(x, *wb)


def _sa_msg(xyz, feats, npoint, radii, nsamples, scale_params):
    fidx = _fps_pallas(jax.lax.stop_gradient(xyz), npoint)
    new_xyz = _index_points(xyz, fidx)
    sg_xyz = jax.lax.stop_gradient(xyz)
    sg_new = jax.lax.stop_gradient(new_xyz)
    outs = []
    s = sg_new.shape[1]
    b, n, _ = xyz.shape
    tbl = xyz if feats is None else jnp.concatenate([xyz, feats], axis=-1)
    c0 = tbl.shape[-1]
    tbl = _pad16(tbl).reshape(b * n, -1)
    for radius, nsample, layers in zip(radii, nsamples, scale_params):
        idx = _ball_query_pallas(radius, nsample, sg_xyz, sg_new)
        flat = (idx + (jnp.arange(b, dtype=jnp.int32) * n)[:, None, None]).reshape(-1)
        rows = _sc_gather_rows(tbl, flat).reshape(b, s, nsample, -1)
        g = rows[..., :3] - new_xyz[:, :, None, :]
        if feats is not None:
            g = jnp.concatenate([g, rows[..., 3:c0]], axis=-1)
        cin = g.shape[-1]
        mb = min(128, s) * nsample
        out = _mlp_pallas(g.reshape(b, s * nsample, cin), layers,
                          pool_k=nsample, m_blk=mb)
        outs.append(out)
    return new_xyz, jnp.concatenate(outs, axis=-1)


def _fp_module(unknown_xyz, known_xyz, unk_feats, known_feats, layers):
    interp = _fp_interp_pallas(jax.lax.stop_gradient(unknown_xyz),
                               jax.lax.stop_gradient(known_xyz), known_feats)
    x = interp if unk_feats is None else jnp.concatenate([interp, unk_feats], axis=-1)
    return _mlp_pallas(x, layers, m_blk=512)


def _final_kernel(feat_ref, tout_ref, fmax_ref):
    f = feat_ref[0]
    tout_ref[0] = jnp.transpose(f, (1, 0))
    fmax_ref[0] = jnp.max(f, axis=0, keepdims=True)


def _final_stage(feat):
    b, n, c = feat.shape
    tout, fmax = pl.pallas_call(
        _final_kernel,
        grid=(b,),
        in_specs=[pl.BlockSpec((1, n, c), lambda i: (i, 0, 0))],
        out_specs=[pl.BlockSpec((1, c, n), lambda i: (i, 0, 0)),
                   pl.BlockSpec((1, 1, c), lambda i: (i, 0, 0))],
        out_shape=[jax.ShapeDtypeStruct((b, c, n), feat.dtype),
                   jax.ShapeDtypeStruct((b, 1, c), feat.dtype)],
    )(feat)
    return tout, fmax[:, 0, :]


def kernel(pointcloud, params):
    xyz = pointcloud[..., :3]
    l_xyz, l_feat = [xyz], [None]
    for (npoint, radii, nsamples, _), sp in zip(SA_SPECS, params['sa']):
        nx, nf = _sa_msg(l_xyz[-1], l_feat[-1], npoint, radii, nsamples, sp)
        l_xyz.append(nx)
        l_feat.append(nf)
    sa_glob = jnp.max(l_feat[-1], axis=1)
    for i in range(-1, -5, -1):
        l_feat[i - 1] = _fp_module(l_xyz[i - 1], l_xyz[i], l_feat[i - 1], l_feat[i], params['fp'][i])
    feat = l_feat[0]
    tfeat, fmax = _final_stage(feat)
    global_feat = jnp.concatenate([fmax, sa_glob], axis=-1)
    return tfeat, global_feat


# speedup vs baseline: 2.5476x; 1.0387x over previous
import functools

import jax
import jax.numpy as jnp
import numpy as np
from jax.experimental import pallas as pl
from jax.experimental.pallas import tpu as pltpu
from jax.experimental.pallas import tpu_sc as plsc

SA_SPECS = [
    (1024, (0.05, 0.1), (16, 32), ((3, 16, 16, 32), (3, 32, 32, 64))),
    (256, (0.1, 0.2), (16, 32), ((99, 64, 64, 128), (99, 64, 96, 128))),
    (64, (0.2, 0.4), (16, 32), ((259, 128, 196, 256), (259, 128, 196, 256))),
    (16, (0.4, 0.8), (16, 32), ((515, 256, 256, 512), (515, 256, 384, 512))),
]
B, N = 4, 8192


def _index_points(points, idx):
    return jax.vmap(lambda p, i: p[i])(points, idx)


def _fps(xyz, npoint):
    xyz = jax.lax.stop_gradient(xyz)
    b, n, _ = xyz.shape
    def step(carry, _):
        dists, farthest = carry
        centroid = _index_points(xyz, farthest[:, None])
        d = jnp.sum((xyz - centroid) ** 2, axis=-1)
        dists = jnp.minimum(dists, d)
        new_far = jnp.argmax(dists, axis=-1).astype(jnp.int32)
        return (dists, new_far), farthest
    init = (jnp.full((b, n), 1e10, jnp.float32), jnp.zeros((b,), jnp.int32))
    _, idxs = jax.lax.scan(step, init, None, length=npoint)
    return jnp.transpose(idxs)


def _ball_query(radius, nsample, xyz, new_xyz):
    b, n, _ = xyz.shape
    s = new_xyz.shape[1]
    d2 = jnp.sum((new_xyz[:, :, None, :] - xyz[:, None, :, :]) ** 2, axis=-1)
    gidx = jnp.broadcast_to(jnp.arange(n, dtype=jnp.int32), (b, s, n))
    gidx = jnp.where(d2 > radius * radius, n, gidx)
    gidx = jnp.sort(gidx, axis=-1)[:, :, :nsample]
    first = gidx[:, :, :1]
    gidx = jnp.where(gidx == n, jnp.broadcast_to(first, gidx.shape), gidx)
    return gidx


def _fps_kernel(npoint, n, b, coordsT_ref, out_ref, dists_ref):
    X = coordsT_ref[:, 0, :]
    Y = coordsT_ref[:, 1, :]
    Z = coordsT_ref[:, 2, :]
    iota = jax.lax.broadcasted_iota(jnp.int32, (b, n), 1)
    dists_ref[...] = jnp.full((b, n), 1e10, jnp.float32)

    def step(j, far):
        out_ref[pl.ds(j, 1), :] = jnp.transpose(far)
        eq = iota == far
        cx = jnp.sum(jnp.where(eq, X, 0.0), axis=1, keepdims=True)
        cy = jnp.sum(jnp.where(eq, Y, 0.0), axis=1, keepdims=True)
        cz = jnp.sum(jnp.where(eq, Z, 0.0), axis=1, keepdims=True)
        dx = X - cx
        dy = Y - cy
        dz = Z - cz
        d = dx * dx + dy * dy + dz * dz
        dists = jnp.minimum(dists_ref[...], d)
        dists_ref[...] = dists
        m = jnp.max(dists, axis=1, keepdims=True)
        return jnp.min(jnp.where(dists == m, iota, n), axis=1, keepdims=True)

    jax.lax.fori_loop(0, npoint, step, jnp.zeros((b, 1), jnp.int32))


def _fps_pallas(xyz, npoint):
    b, n, _ = xyz.shape
    coordsT = jnp.transpose(xyz, (0, 2, 1))
    fn = functools.partial(_fps_kernel, npoint, n, b)
    out = pl.pallas_call(
        fn,
        grid=(1,),
        in_specs=[pl.BlockSpec((b, 3, n), lambda i: (0, 0, 0))],
        out_specs=pl.BlockSpec((npoint, b), lambda i: (0, 0)),
        out_shape=jax.ShapeDtypeStruct((npoint, b), jnp.int32),
        scratch_shapes=[pltpu.VMEM((b, n), jnp.float32)],
    )(coordsT)
    return jnp.transpose(out)


def _bq_kernel(n_actual, radii2, nsamples, new_ref, xyzT_ref, *out_refs):
    new = new_ref[0]
    cx, cy, cz = new[:, 0:1], new[:, 1:2], new[:, 2:3]
    X = xyzT_ref[0]
    dx = cx - X[0:1, :]
    dy = cy - X[1:2, :]
    dz = cz - X[2:3, :]
    d2 = dx * dx + dy * dy + dz * dz
    iota = jax.lax.broadcasted_iota(jnp.int32, d2.shape, 1)
    for r2, nsample, out_ref in zip(radii2, nsamples, out_refs):
        key = jnp.where(d2 > r2, n_actual, iota)
        cols = []
        for _ in range(nsample):
            m = jnp.min(key, axis=1, keepdims=True)
            cols.append(m)
            key = jnp.where(key == m, n_actual, key)
        out = jnp.concatenate(cols, axis=1)
        first = out[:, 0:1]
        out_ref[0] = jnp.where(out == n_actual, first, out)


def _ball_query_pallas(radii, nsamples, xyz, new_xyz):
    b, n, _ = xyz.shape
    s = new_xyz.shape[1]
    xyzT = jnp.transpose(xyz, (0, 2, 1))
    bs = min(128, s)
    r2s = tuple(np.float32(r * r) for r in radii)
    fn = functools.partial(_bq_kernel, n, r2s, tuple(nsamples))
    return pl.pallas_call(
        fn,
        grid=(b, s // bs),
        in_specs=[pl.BlockSpec((1, bs, 3), lambda i, j: (i, j, 0)),
                  pl.BlockSpec((1, 3, n), lambda i, j: (i, 0, 0))],
        out_specs=[pl.BlockSpec((1, bs, k), lambda i, j: (i, j, 0))
                   for k in nsamples],
        out_shape=[jax.ShapeDtypeStruct((b, s, k), jnp.int32)
                   for k in nsamples],
    )(new_xyz, xyzT)


def _interp_kernel(s_actual, unk_ref, knownT_ref, feats_ref, out_ref):
    unk = unk_ref[0]
    Kt = knownT_ref[0]
    dx = unk[:, 0:1] - Kt[0:1, :]
    dy = unk[:, 1:2] - Kt[1:2, :]
    dz = unk[:, 2:3] - Kt[2:3, :]
    d2 = dx * dx + dy * dy + dz * dz
    iota = jax.lax.broadcasted_iota(jnp.int32, d2.shape, 1)
    dvals, idxs = [], []
    work = d2
    for _ in range(3):
        m = jnp.min(work, axis=1, keepdims=True)
        ix = jnp.min(jnp.where(work == m, iota, s_actual), axis=1, keepdims=True)
        dvals.append(m)
        idxs.append(ix)
        work = jnp.where(iota == ix, jnp.float32(jnp.inf), work)
    ws = [1.0 / (m + 1e-8) for m in dvals]
    wsum = ws[0] + ws[1] + ws[2]
    W = jnp.zeros_like(d2)
    for wk, ix in zip(ws, idxs):
        W = W + jnp.where(iota == ix, wk / wsum, 0.0)
    out_ref[0] = jax.lax.dot_general(
        W, feats_ref[0], (((1,), (0,)), ((), ())),
        preferred_element_type=jnp.float32,
        precision=jax.lax.Precision.HIGHEST)


def _fp_interp_pallas(unknown_xyz, known_xyz, known_feats):
    b, nu, _ = unknown_xyz.shape
    s = known_xyz.shape[1]
    c = known_feats.shape[2]
    knownT = jnp.transpose(known_xyz, (0, 2, 1))
    bu = min(256, nu)
    fn = functools.partial(_interp_kernel, s)
    return pl.pallas_call(
        fn,
        grid=(b, nu // bu),
        in_specs=[pl.BlockSpec((1, bu, 3), lambda i, j: (i, j, 0)),
                  pl.BlockSpec((1, 3, s), lambda i, j: (i, 0, 0)),
                  pl.BlockSpec((1, s, c), lambda i, j: (i, 0, 0))],
        out_specs=pl.BlockSpec((1, bu, c), lambda i, j: (i, j, 0)),
        out_shape=jax.ShapeDtypeStruct((b, nu, c), jnp.float32),
    )(unknown_xyz, knownT, known_feats)


def _sc_gather_rows(table, idx_flat):
    m = idx_flat.shape[0]
    d = table.shape[1]
    info = plsc.get_sparse_core_info()
    ncores = info.num_cores
    nw = ncores * info.num_subcores
    bpw = m // nw
    chunk = min(128, bpw)
    nchunk = bpw // chunk
    idx3 = idx_flat.reshape(nw, nchunk, chunk)

    @functools.partial(
        pl.kernel,
        mesh=plsc.VectorSubcoreMesh(core_axis_name="c", subcore_axis_name="s"),
        out_type=jax.ShapeDtypeStruct((m, d), jnp.float32),
        compiler_params=pltpu.CompilerParams(use_tc_tiling_on_sc=False),
        scratch_types=[pltpu.VMEM((nchunk, chunk), jnp.int32),
                       pltpu.VMEM((bpw, d), jnp.float32),
                       pltpu.SemaphoreType.DMA],
    )
    def gk(table_hbm, idx_hbm, out_hbm, idx_v, rows_v, sem):
        wid = jax.lax.axis_index("s") * ncores + jax.lax.axis_index("c")
        pltpu.sync_copy(idx_hbm.at[wid], idx_v)

        def start(j, carry):
            pltpu.make_async_copy(table_hbm.at[idx_v.at[j]],
                                  rows_v.at[pl.ds(j * chunk, chunk)], sem).start()
            return carry

        jax.lax.fori_loop(0, nchunk, start, 0)

        def drain(j, carry):
            pltpu.make_async_copy(table_hbm.at[idx_v.at[j]],
                                  rows_v.at[pl.ds(j * chunk, chunk)], sem).wait()
            return carry

        jax.lax.fori_loop(0, nchunk, drain, 0)
        pltpu.sync_copy(rows_v, out_hbm.at[pl.ds(wid * bpw, bpw)])

    return gk(table, idx3)


def _pad16(x):
    c = x.shape[-1]
    pad = (-c) % 16
    if pad:
        x = jnp.concatenate([x, jnp.zeros(x.shape[:-1] + (pad,), x.dtype)], axis=-1)
    return x


def _mlp_kernel(nlayers, pool_k, x_ref, *refs):
    out_ref = refs[-1]
    x = x_ref[0]
    for li in range(nlayers):
        w = refs[2 * li][...]
        bias = refs[2 * li + 1][...]
        x = jax.lax.dot_general(x, w, (((1,), (0,)), ((), ())),
                                preferred_element_type=jnp.float32,
                                precision=jax.lax.Precision.HIGHEST)
        x = jnp.maximum(x + bias, 0.0)
    if pool_k is not None:
        m, c = x.shape
        x = jnp.max(x.reshape(m // pool_k, pool_k, c), axis=1)
    out_ref[0] = x


def _mlp_pallas(x, layers, pool_k=None, m_blk=512):
    b, m, cin = x.shape
    mb = min(m_blk, m)
    cout = layers[-1][0].shape[0]
    mout = m // pool_k if pool_k is not None else m
    mob = mb // pool_k if pool_k is not None else mb
    wb = []
    in_specs = [pl.BlockSpec((1, mb, cin), lambda i, j: (i, j, 0))]
    for W, bias in layers:
        wT = jnp.transpose(W)
        wb.extend([wT, bias[None, :]])
        in_specs.append(pl.BlockSpec(wT.shape, lambda i, j: (0, 0)))
        in_specs.append(pl.BlockSpec((1, bias.shape[0]), lambda i, j: (0, 0)))
    fn = functools.partial(_mlp_kernel, len(layers), pool_k)
    return pl.pallas_call(
        fn,
        grid=(b, m // mb),
        in_specs=in_specs,
        out_specs=pl.BlockSpec((1, mob, cout), lambda i, j: (i, j, 0)),
        out_shape=jax.ShapeDtypeStruct((b, mout, cout), jnp.float32),
    )(x, *wb)


def _sa_msg(xyz, feats, npoint, radii, nsamples, scale_params):
    fidx = _fps_pallas(jax.lax.stop_gradient(xyz), npoint)
    new_xyz = _index_points(xyz, fidx)
    sg_xyz = jax.lax.stop_gradient(xyz)
    sg_new = jax.lax.stop_gradient(new_xyz)
    outs = []
    s = sg_new.shape[1]
    b, n, _ = xyz.shape
    tbl = xyz if feats is None else jnp.concatenate([xyz, feats], axis=-1)
    c0 = tbl.shape[-1]
    tbl = _pad16(tbl).reshape(b * n, -1)
    idxs = _ball_query_pallas(radii, nsamples, sg_xyz, sg_new)
    for idx, nsample, layers in zip(idxs, nsamples, scale_params):
        flat = (idx + (jnp.arange(b, dtype=jnp.int32) * n)[:, None, None]).reshape(-1)
        rows = _sc_gather_rows(tbl, flat).reshape(b, s, nsample, -1)
        g = rows[..., :3] - new_xyz[:, :, None, :]
        if feats is not None:
            g = jnp.concatenate([g, rows[..., 3:c0]], axis=-1)
        cin = g.shape[-1]
        mb = min(128, s) * nsample
        out = _mlp_pallas(g.reshape(b, s * nsample, cin), layers,
                          pool_k=nsample, m_blk=mb)
        outs.append(out)
    return new_xyz, jnp.concatenate(outs, axis=-1)


def _fp_module(unknown_xyz, known_xyz, unk_feats, known_feats, layers):
    interp = _fp_interp_pallas(jax.lax.stop_gradient(unknown_xyz),
                               jax.lax.stop_gradient(known_xyz), known_feats)
    x = interp if unk_feats is None else jnp.concatenate([interp, unk_feats], axis=-1)
    return _mlp_pallas(x, layers, m_blk=512)


def _final_kernel(feat_ref, tout_ref, fmax_ref):
    f = feat_ref[0]
    tout_ref[0] = jnp.transpose(f, (1, 0))
    fmax_ref[0] = jnp.max(f, axis=0, keepdims=True)


def _final_stage(feat):
    b, n, c = feat.shape
    tout, fmax = pl.pallas_call(
        _final_kernel,
        grid=(b,),
        in_specs=[pl.BlockSpec((1, n, c), lambda i: (i, 0, 0))],
        out_specs=[pl.BlockSpec((1, c, n), lambda i: (i, 0, 0)),
                   pl.BlockSpec((1, 1, c), lambda i: (i, 0, 0))],
        out_shape=[jax.ShapeDtypeStruct((b, c, n), feat.dtype),
                   jax.ShapeDtypeStruct((b, 1, c), feat.dtype)],
    )(feat)
    return tout, fmax[:, 0, :]


def kernel(pointcloud, params):
    xyz = pointcloud[..., :3]
    l_xyz, l_feat = [xyz], [None]
    for (npoint, radii, nsamples, _), sp in zip(SA_SPECS, params['sa']):
        nx, nf = _sa_msg(l_xyz[-1], l_feat[-1], npoint, radii, nsamples, sp)
        l_xyz.append(nx)
        l_feat.append(nf)
    sa_glob = jnp.max(l_feat[-1], axis=1)
    for i in range(-1, -5, -1):
        l_feat[i - 1] = _fp_module(l_xyz[i - 1], l_xyz[i], l_feat[i - 1], l_feat[i], params['fp'][i])
    feat = l_feat[0]
    tfeat, fmax = _final_stage(feat)
    global_feat = jnp.concatenate([fmax, sa_glob], axis=-1)
    return tfeat, global_feat


# speedup vs baseline: 2.5483x; 1.0003x over previous
import functools

import jax
import jax.numpy as jnp
import numpy as np
from jax.experimental import pallas as pl
from jax.experimental.pallas import tpu as pltpu
from jax.experimental.pallas import tpu_sc as plsc

SA_SPECS = [
    (1024, (0.05, 0.1), (16, 32), ((3, 16, 16, 32), (3, 32, 32, 64))),
    (256, (0.1, 0.2), (16, 32), ((99, 64, 64, 128), (99, 64, 96, 128))),
    (64, (0.2, 0.4), (16, 32), ((259, 128, 196, 256), (259, 128, 196, 256))),
    (16, (0.4, 0.8), (16, 32), ((515, 256, 256, 512), (515, 256, 384, 512))),
]
B, N = 4, 8192


def _index_points(points, idx):
    return jax.vmap(lambda p, i: p[i])(points, idx)


def _fps_kernel(npoint, n, b, coordsT_ref, out_ref, dists_ref):
    X = coordsT_ref[:, 0, :]
    Y = coordsT_ref[:, 1, :]
    Z = coordsT_ref[:, 2, :]
    iota = jax.lax.broadcasted_iota(jnp.int32, (b, n), 1)
    dists_ref[...] = jnp.full((b, n), 1e10, jnp.float32)

    def step(j, far):
        out_ref[pl.ds(j, 1), :] = jnp.transpose(far)
        eq = iota == far
        cx = jnp.sum(jnp.where(eq, X, 0.0), axis=1, keepdims=True)
        cy = jnp.sum(jnp.where(eq, Y, 0.0), axis=1, keepdims=True)
        cz = jnp.sum(jnp.where(eq, Z, 0.0), axis=1, keepdims=True)
        dx = X - cx
        dy = Y - cy
        dz = Z - cz
        d = dx * dx + dy * dy + dz * dz
        dists = jnp.minimum(dists_ref[...], d)
        dists_ref[...] = dists
        m = jnp.max(dists, axis=1, keepdims=True)
        return jnp.min(jnp.where(dists == m, iota, n), axis=1, keepdims=True)

    jax.lax.fori_loop(0, npoint, step, jnp.zeros((b, 1), jnp.int32))


def _fps_pallas(xyz, npoint):
    b, n, _ = xyz.shape
    coordsT = jnp.transpose(xyz, (0, 2, 1))
    fn = functools.partial(_fps_kernel, npoint, n, b)
    out = pl.pallas_call(
        fn,
        grid=(1,),
        in_specs=[pl.BlockSpec((b, 3, n), lambda i: (0, 0, 0))],
        out_specs=pl.BlockSpec((npoint, b), lambda i: (0, 0)),
        out_shape=jax.ShapeDtypeStruct((npoint, b), jnp.int32),
        scratch_shapes=[pltpu.VMEM((b, n), jnp.float32)],
    )(coordsT)
    return jnp.transpose(out)


def _bq_kernel(n_actual, radii2, nsamples, new_ref, xyzT_ref, *out_refs):
    new = new_ref[0]
    cx, cy, cz = new[:, 0:1], new[:, 1:2], new[:, 2:3]
    X = xyzT_ref[0]
    dx = cx - X[0:1, :]
    dy = cy - X[1:2, :]
    dz = cz - X[2:3, :]
    d2 = dx * dx + dy * dy + dz * dz
    iota = jax.lax.broadcasted_iota(jnp.int32, d2.shape, 1)
    for r2, nsample, out_ref in zip(radii2, nsamples, out_refs):
        key = jnp.where(d2 > r2, n_actual, iota)
        cols = []
        for _ in range(nsample):
            m = jnp.min(key, axis=1, keepdims=True)
            cols.append(m)
            key = jnp.where(key == m, n_actual, key)
        out = jnp.concatenate(cols, axis=1)
        first = out[:, 0:1]
        out_ref[0] = jnp.where(out == n_actual, first, out)


def _ball_query_pallas(radii, nsamples, xyz, new_xyz):
    b, n, _ = xyz.shape
    s = new_xyz.shape[1]
    xyzT = jnp.transpose(xyz, (0, 2, 1))
    bs = min(128, s)
    r2s = tuple(np.float32(r * r) for r in radii)
    fn = functools.partial(_bq_kernel, n, r2s, tuple(nsamples))
    return pl.pallas_call(
        fn,
        grid=(b, s // bs),
        in_specs=[pl.BlockSpec((1, bs, 3), lambda i, j: (i, j, 0)),
                  pl.BlockSpec((1, 3, n), lambda i, j: (i, 0, 0))],
        out_specs=[pl.BlockSpec((1, bs, k), lambda i, j: (i, j, 0))
                   for k in nsamples],
        out_shape=[jax.ShapeDtypeStruct((b, s, k), jnp.int32)
                   for k in nsamples],
    )(new_xyz, xyzT)


def _interp_kernel(s_actual, unk_ref, knownT_ref, feats_ref, out_ref):
    unk = unk_ref[0]
    Kt = knownT_ref[0]
    dx = unk[:, 0:1] - Kt[0:1, :]
    dy = unk[:, 1:2] - Kt[1:2, :]
    dz = unk[:, 2:3] - Kt[2:3, :]
    d2 = dx * dx + dy * dy + dz * dz
    iota = jax.lax.broadcasted_iota(jnp.int32, d2.shape, 1)
    dvals, idxs = [], []
    work = d2
    for _ in range(3):
        m = jnp.min(work, axis=1, keepdims=True)
        ix = jnp.min(jnp.where(work == m, iota, s_actual), axis=1, keepdims=True)
        dvals.append(m)
        idxs.append(ix)
        work = jnp.where(iota == ix, jnp.float32(jnp.inf), work)
    ws = [1.0 / (m + 1e-8) for m in dvals]
    wsum = ws[0] + ws[1] + ws[2]
    W = jnp.zeros_like(d2)
    for wk, ix in zip(ws, idxs):
        W = W + jnp.where(iota == ix, wk / wsum, 0.0)
    out_ref[0] = jax.lax.dot_general(
        W, feats_ref[0], (((1,), (0,)), ((), ())),
        preferred_element_type=jnp.float32,
        precision=jax.lax.Precision.HIGHEST)


def _fp_interp_pallas(unknown_xyz, known_xyz, known_feats):
    b, nu, _ = unknown_xyz.shape
    s = known_xyz.shape[1]
    c = known_feats.shape[2]
    knownT = jnp.transpose(known_xyz, (0, 2, 1))
    bu = min(256, nu)
    fn = functools.partial(_interp_kernel, s)
    return pl.pallas_call(
        fn,
        grid=(b, nu // bu),
        in_specs=[pl.BlockSpec((1, bu, 3), lambda i, j: (i, j, 0)),
                  pl.BlockSpec((1, 3, s), lambda i, j: (i, 0, 0)),
                  pl.BlockSpec((1, s, c), lambda i, j: (i, 0, 0))],
        out_specs=pl.BlockSpec((1, bu, c), lambda i, j: (i, j, 0)),
        out_shape=jax.ShapeDtypeStruct((b, nu, c), jnp.float32),
    )(unknown_xyz, knownT, known_feats)


def _sc_gather_rows(table, idx_flat):
    m = idx_flat.shape[0]
    d = table.shape[1]
    info = plsc.get_sparse_core_info()
    ncores = info.num_cores
    nw = ncores * info.num_subcores
    bpw = m // nw
    chunk = min(128, bpw)
    nchunk = bpw // chunk
    idx3 = idx_flat.reshape(nw, nchunk, chunk)

    @functools.partial(
        pl.kernel,
        mesh=plsc.VectorSubcoreMesh(core_axis_name="c", subcore_axis_name="s"),
        out_type=jax.ShapeDtypeStruct((m, d), jnp.float32),
        compiler_params=pltpu.CompilerParams(use_tc_tiling_on_sc=False),
        scratch_types=[pltpu.VMEM((nchunk, chunk), jnp.int32),
                       pltpu.VMEM((bpw, d), jnp.float32),
                       pltpu.SemaphoreType.DMA],
    )
    def gk(table_hbm, idx_hbm, out_hbm, idx_v, rows_v, sem):
        wid = jax.lax.axis_index("s") * ncores + jax.lax.axis_index("c")
        pltpu.sync_copy(idx_hbm.at[wid], idx_v)

        def start(j, carry):
            pltpu.make_async_copy(table_hbm.at[idx_v.at[j]],
                                  rows_v.at[pl.ds(j * chunk, chunk)], sem).start()
            return carry

        jax.lax.fori_loop(0, nchunk, start, 0)

        def drain(j, carry):
            pltpu.make_async_copy(table_hbm.at[idx_v.at[j]],
                                  rows_v.at[pl.ds(j * chunk, chunk)], sem).wait()
            return carry

        jax.lax.fori_loop(0, nchunk, drain, 0)
        pltpu.sync_copy(rows_v, out_hbm.at[pl.ds(wid * bpw, bpw)])

    return gk(table, idx3)


def _pad16(x):
    c = x.shape[-1]
    pad = (-c) % 16
    if pad:
        x = jnp.concatenate([x, jnp.zeros(x.shape[:-1] + (pad,), x.dtype)], axis=-1)
    return x


def _mlp_kernel(nlayers, pool_k, x_ref, *refs):
    out_ref = refs[-1]
    x = x_ref[0]
    for li in range(nlayers):
        w = refs[2 * li][...]
        bias = refs[2 * li + 1][...]
        x = jax.lax.dot_general(x, w, (((1,), (0,)), ((), ())),
                                preferred_element_type=jnp.float32,
                                precision=jax.lax.Precision.HIGHEST)
        x = jnp.maximum(x + bias, 0.0)
    if pool_k is not None:
        m, c = x.shape
        x = jnp.max(x.reshape(m // pool_k, pool_k, c), axis=1)
    out_ref[0] = x


def _mlp_pallas(x, layers, pool_k=None, m_blk=512):
    b, m, cin = x.shape
    mb = min(m_blk, m)
    cout = layers[-1][0].shape[0]
    mout = m // pool_k if pool_k is not None else m
    mob = mb // pool_k if pool_k is not None else mb
    wb = []
    in_specs = [pl.BlockSpec((1, mb, cin), lambda i, j: (i, j, 0))]
    for W, bias in layers:
        wT = jnp.transpose(W)
        wb.extend([wT, bias[None, :]])
        in_specs.append(pl.BlockSpec(wT.shape, lambda i, j: (0, 0)))
        in_specs.append(pl.BlockSpec((1, bias.shape[0]), lambda i, j: (0, 0)))
    fn = functools.partial(_mlp_kernel, len(layers), pool_k)
    return pl.pallas_call(
        fn,
        grid=(b, m // mb),
        in_specs=in_specs,
        out_specs=pl.BlockSpec((1, mob, cout), lambda i, j: (i, j, 0)),
        out_shape=jax.ShapeDtypeStruct((b, mout, cout), jnp.float32),
    )(x, *wb)


def _sa_msg(xyz, feats, npoint, radii, nsamples, scale_params):
    fidx = _fps_pallas(jax.lax.stop_gradient(xyz), npoint)
    new_xyz = _index_points(xyz, fidx)
    sg_xyz = jax.lax.stop_gradient(xyz)
    sg_new = jax.lax.stop_gradient(new_xyz)
    outs = []
    s = sg_new.shape[1]
    b, n, _ = xyz.shape
    tbl = xyz if feats is None else jnp.concatenate([xyz, feats], axis=-1)
    c0 = tbl.shape[-1]
    tbl = _pad16(tbl).reshape(b * n, -1)
    idxs = _ball_query_pallas(radii, nsamples, sg_xyz, sg_new)
    for idx, nsample, layers in zip(idxs, nsamples, scale_params):
        flat = (idx + (jnp.arange(b, dtype=jnp.int32) * n)[:, None, None]).reshape(-1)
        rows = _sc_gather_rows(tbl, flat).reshape(b, s, nsample, -1)
        g = rows[..., :3] - new_xyz[:, :, None, :]
        if feats is not None:
            g = jnp.concatenate([g, rows[..., 3:c0]], axis=-1)
        cin = g.shape[-1]
        mb = min(128, s) * nsample
        out = _mlp_pallas(g.reshape(b, s * nsample, cin), layers,
                          pool_k=nsample, m_blk=mb)
        outs.append(out)
    return new_xyz, jnp.concatenate(outs, axis=-1)


def _fp_module(unknown_xyz, known_xyz, unk_feats, known_feats, layers):
    interp = _fp_interp_pallas(jax.lax.stop_gradient(unknown_xyz),
                               jax.lax.stop_gradient(known_xyz), known_feats)
    x = interp if unk_feats is None else jnp.concatenate([interp, unk_feats], axis=-1)
    return _mlp_pallas(x, layers, m_blk=512)


def _final_kernel(feat_ref, tout_ref, fmax_ref):
    f = feat_ref[0]
    tout_ref[0] = jnp.transpose(f, (1, 0))
    fmax_ref[0] = jnp.max(f, axis=0, keepdims=True)


def _final_stage(feat):
    b, n, c = feat.shape
    tout, fmax = pl.pallas_call(
        _final_kernel,
        grid=(b,),
        in_specs=[pl.BlockSpec((1, n, c), lambda i: (i, 0, 0))],
        out_specs=[pl.BlockSpec((1, c, n), lambda i: (i, 0, 0)),
                   pl.BlockSpec((1, 1, c), lambda i: (i, 0, 0))],
        out_shape=[jax.ShapeDtypeStruct((b, c, n), feat.dtype),
                   jax.ShapeDtypeStruct((b, 1, c), feat.dtype)],
    )(feat)
    return tout, fmax[:, 0, :]


def kernel(pointcloud, params):
    xyz = pointcloud[..., :3]
    l_xyz, l_feat = [xyz], [None]
    for (npoint, radii, nsamples, _), sp in zip(SA_SPECS, params['sa']):
        nx, nf = _sa_msg(l_xyz[-1], l_feat[-1], npoint, radii, nsamples, sp)
        l_xyz.append(nx)
        l_feat.append(nf)
    sa_glob = jnp.max(l_feat[-1], axis=1)
    for i in range(-1, -5, -1):
        l_feat[i - 1] = _fp_module(l_xyz[i - 1], l_xyz[i], l_feat[i - 1], l_feat[i], params['fp'][i])
    feat = l_feat[0]
    tfeat, fmax = _final_stage(feat)
    global_feat = jnp.concatenate([fmax, sa_glob], axis=-1)
    return tfeat, global_feat
